# baseline (speedup 1.0000x reference)
import functools

import jax
import jax.numpy as jnp
from jax import lax
from jax.experimental import pallas as pl
from jax.experimental.pallas import tpu as pltpu
from jax.experimental.pallas import tpu_sc as plsc

DIM = 33
TBL = DIM * DIM * DIM
TBL_PAD = 35944
NLUT = 3 * TBL_PAD
BINSIZE = 1.000001 / (DIM - 1)
INV_BIN = float(1.0 / BINSIZE)


def _pack_lut(lut):
  flat = lut.reshape(3, TBL) - jnp.float32(0.5)
  u = lax.bitcast_convert_type(flat, jnp.uint32).astype(jnp.int64)
  u = jnp.pad(u, ((0, 0), (0, 1)))
  rnd = lambda t: (t + 0x7FFF + ((t >> 16) & 1)) >> 16
  rb1 = rnd(u[:, 1:])
  rb0 = rnd(u[:, :-1])
  word = ((rb0 << 16) | rb1).astype(jnp.uint32)
  word = jnp.pad(word, ((0, 0), (0, TBL_PAD - TBL)))
  return lax.bitcast_convert_type(word, jnp.int32).reshape(-1)


@functools.lru_cache(maxsize=None)
def _build(n_batch, pixels):
  info = plsc.get_sparse_core_info()
  NC, NS, L = info.num_cores, info.num_subcores, info.num_lanes
  NW = NC * NS
  ppw = pixels // NW
  C = 1024
  steps = ppw // C
  T = n_batch * steps
  chan_stride = pixels
  batch_stride = 3 * pixels

  mesh = plsc.VectorSubcoreMesh(core_axis_name="c", subcore_axis_name="s")

  buf = lambda: pltpu.VMEM((C,), jnp.float32)
  HI = jnp.int32(-65536)

  @functools.partial(
      pl.kernel,
      mesh=mesh,
      compiler_params=pltpu.CompilerParams(needs_layout_passes=False),
      out_type=jax.ShapeDtypeStruct((n_batch * 3 * pixels,), jnp.float32),
      scratch_types=[
          pltpu.VMEM((NLUT,), jnp.int32),
          buf(), buf(), buf(), buf(), buf(), buf(),
          buf(), buf(), buf(), buf(), buf(), buf(),
          pltpu.SemaphoreType.DMA, pltpu.SemaphoreType.DMA,
          pltpu.SemaphoreType.DMA, pltpu.SemaphoreType.DMA,
      ],
  )
  def sc_kernel(lut_hbm, x_hbm, out_hbm, lut_v,
                rvA, gvA, bvA, rvB, gvB, bvB,
                orA, ogA, obA, orB, ogB, obB,
                siA, siB, soA, soB):
    wid = lax.axis_index("s") * NC + lax.axis_index("c")
    pltpu.sync_copy(lut_hbm, lut_v)
    base0 = wid * ppw

    def t_start(t):
      b = t // steps
      s = t - b * steps
      return b * batch_stride + base0 + s * C

    def issue_in(t, rv, gv, bv, sem):
      start = t_start(t)
      pltpu.async_copy(x_hbm.at[pl.ds(start, C)], rv, sem)
      pltpu.async_copy(x_hbm.at[pl.ds(start + chan_stride, C)], gv, sem)
      pltpu.async_copy(x_hbm.at[pl.ds(start + 2 * chan_stride, C)], bv, sem)

    def issue_out(t, orv, ogv, obv, sem):
      start = t_start(t)
      pltpu.async_copy(orv, out_hbm.at[pl.ds(start, C)], sem)
      pltpu.async_copy(ogv, out_hbm.at[pl.ds(start + chan_stride, C)], sem)
      pltpu.async_copy(obv, out_hbm.at[pl.ds(start + 2 * chan_stride, C)], sem)

    def drain3(sem, dst):
      for _ in range(3):
        pltpu.make_async_copy(x_hbm.at[pl.ds(0, C)], dst, sem).wait()

    def compute(rv, gv, bv, orv, ogv, obv):
      def half(off, rv, gv, bv, orv, ogv, obv):
        rs = rv[pl.ds(off, L)] * INV_BIN
        gs = gv[pl.ds(off, L)] * INV_BIN
        bs = bv[pl.ds(off, L)] * INV_BIN
        ri = rs.astype(jnp.int32)
        gi = gs.astype(jnp.int32)
        bi = bs.astype(jnp.int32)
        rd = rs - ri.astype(jnp.float32)
        gd = gs - gi.astype(jnp.float32)
        bd = bs - bi.astype(jnp.float32)
        gd1 = 1.0 - gd
        bd1 = 1.0 - bd
        w = (gd1 * bd1, gd * bd1, gd1 * bd, gd * bd)
        base = ri + gi * DIM + bi * (DIM * DIM)
        offs = (0, DIM, DIM * DIM, DIM * DIM + DIM)
        pk = [plsc.load_gather(lut_v, [base + (c * TBL_PAD + o)])
              for c in range(3) for o in offs]
        pb = [plsc.bitcast(q, jnp.bfloat16) for q in pk]
        wb = [plsc.pack(t, t, format=plsc.PackFormat.INTERLEAVED) for t in w]
        res = []
        for c in range(3):
          j = 4 * c
          t = ((wb[0] * pb[j] + wb[1] * pb[j + 1])
               + (wb[2] * pb[j + 2] + wb[3] * pb[j + 3]))
          vb, va = plsc.unpack(t, format=plsc.PackFormat.INTERLEAVED)
          res.append((va + rd * (vb - va)) + 0.5)
        orv[pl.ds(off, L)] = res[0]
        ogv[pl.ds(off, L)] = res[1]
        obv[pl.ds(off, L)] = res[2]

      def vec(i, c2):
        off = i * (2 * L)
        half(off, rv, gv, bv, orv, ogv, obv)
        half(off + L, rv, gv, bv, orv, ogv, obv)
        return c2

      lax.fori_loop(0, C // (2 * L), vec, 0)

    issue_in(0, rvA, gvA, bvA, siA)
    issue_in(1, rvB, gvB, bvB, siB)

    def body(k, carry):
      tA = 2 * k
      tB = 2 * k + 1

      drain3(siA, rvA)
      @pl.when(k > 0)
      def _():
        drain3(soA, orA)
      compute(rvA, gvA, bvA, orA, ogA, obA)
      issue_out(tA, orA, ogA, obA, soA)
      @pl.when(tA + 2 < T)
      def _():
        issue_in(tA + 2, rvA, gvA, bvA, siA)

      drain3(siB, rvB)
      @pl.when(k > 0)
      def _():
        drain3(soB, orB)
      compute(rvB, gvB, bvB, orB, ogB, obB)
      issue_out(tB, orB, ogB, obB, soB)
      @pl.when(tB + 2 < T)
      def _():
        issue_in(tB + 2, rvB, gvB, bvB, siB)
      return carry

    lax.fori_loop(0, T // 2, body, 0)
    drain3(soA, orA)
    drain3(soB, orB)

  return sc_kernel


def kernel(lut_count, lut, x):
  n_batch = x.shape[0]
  pixels = x.shape[2] * x.shape[3]
  fn = _build(n_batch, pixels)
  out = fn(_pack_lut(lut), x.reshape(-1))
  return (lut, out.reshape(x.shape))
